# baseline (speedup 1.0000x reference)
import functools

import jax
import jax.numpy as jnp
from jax import lax
from jax.experimental import pallas as pl
from jax.experimental.pallas import tpu as pltpu
from jax.experimental.pallas import tpu_sc as plsc

N = 10000
E = 320000
D = 128
H = 2 * D
L = 4
NLIN = 2
EPS = 1e-7

NTILES = 16
EPT = E // NTILES
K = 128
NCHUNKS = E // K
NCH_BASE = NCHUNKS // NTILES
XTRA = NCHUNKS - NCH_BASE * NTILES
NBUF = 3
ROWS_PT = 624
ROWS_TAIL = N - NTILES * ROWS_PT

@functools.cache
def _sc_agg_kernel():
  mesh = plsc.VectorSubcoreMesh(core_axis_name="c", subcore_axis_name="s",
                                num_cores=2, num_subcores=NTILES)

  @functools.partial(
      pl.kernel,
      out_type=jax.ShapeDtypeStruct((2 * N, D), jnp.float32),
      mesh=mesh,
      scratch_types=[
        pltpu.VMEM((NBUF, K), jnp.int32),
        pltpu.VMEM((NBUF, K), jnp.int32),
        pltpu.VMEM((NBUF, K), jnp.int32),
        pltpu.VMEM((NBUF, K, D), jnp.float32),
        pltpu.VMEM_SHARED((N, D), jnp.float32),
        pltpu.SemaphoreType.DMA((NBUF,)),
        pltpu.SemaphoreType.DMA((NBUF,)),
        pltpu.SemaphoreType.DMA((NBUF,)),
        pltpu.SemaphoreType.DMA((NBUF,)),
      ],
  )
  def _sc_agg(tab_hbm, src_hbm, dst_hbm, zeros_hbm, out_hbm,
              idxd_v, gsrc_v, sdst_v, rows_v,
              acc_sh, sem_is, sem_id, sem_g, sem_w):
    c = lax.axis_index("c")
    s = lax.axis_index("s")
    pltpu.sync_copy(zeros_hbm.at[pl.ds(0, ROWS_PT)],
                    acc_sh.at[pl.ds(s * ROWS_PT, ROWS_PT)])

    @pl.when(s == NTILES - 1)
    def _zero_tail():
      pltpu.sync_copy(zeros_hbm.at[pl.ds(0, ROWS_TAIL)],
                      acc_sh.at[pl.ds(NTILES * ROWS_PT, ROWS_TAIL)])

    plsc.subcore_barrier()

    nch = NCH_BASE + jnp.where(s < XTRA, 1, 0)
    base0 = s * (NCH_BASE * K) + jnp.minimum(s, XTRA) * K
    off = c * N

    def start_idx(j, b):
      o = base0 + j * K
      pltpu.async_copy(src_hbm.at[pl.ds(o, K)], gsrc_v.at[b], sem_is.at[b])
      pltpu.async_copy(dst_hbm.at[pl.ds(o, K)], idxd_v.at[b], sem_id.at[b])

    def wait_idx(j, b):
      o = base0 + j * K
      pltpu.make_async_copy(src_hbm.at[pl.ds(o, K)], gsrc_v.at[b],
                            sem_is.at[b]).wait()
      pltpu.make_async_copy(dst_hbm.at[pl.ds(o, K)], idxd_v.at[b],
                            sem_id.at[b]).wait()

    def copy_idx(b):
      gs, sd = gsrc_v.at[b], sdst_v.at[b]
      id_ = idxd_v.at[b]
      for k in range(K // 16):
        sl = pl.ds(k * 16, 16)
        gs[sl] = gs[sl] + off
        sd[sl] = id_[sl]

    def start_gather(b):
      pltpu.async_copy(tab_hbm.at[gsrc_v.at[b]], rows_v.at[b], sem_g.at[b])

    def wait_gather(b):
      pltpu.make_async_copy(tab_hbm.at[gsrc_v.at[b]], rows_v.at[b],
                            sem_g.at[b]).wait()

    def start_scat(b):
      pltpu.async_copy(rows_v.at[b], acc_sh.at[sdst_v.at[b]], sem_w.at[b],
                       add=True)

    def wait_scat(b):
      pltpu.make_async_copy(rows_v.at[b], acc_sh.at[sdst_v.at[b]],
                            sem_w.at[b]).wait()

    for jj in range(NBUF):
      start_idx(jj, jj)

    @pl.loop(0, nch)
    def _chunk(j):
      b = lax.rem(j, NBUF)
      pb = lax.rem(j + NBUF - 1, NBUF)
      wait_idx(j, b)

      @pl.when(j >= NBUF)
      def _():
        wait_scat(b)

      copy_idx(b)
      start_gather(b)

      @pl.when(j >= 1)
      def _():
        wait_gather(pb)
        start_scat(pb)

        @pl.when(j - 1 + NBUF < nch)
        def _():
          start_idx(j - 1 + NBUF, pb)

    last = lax.rem(nch - 1, NBUF)
    wait_gather(last)
    start_scat(last)
    for bb in range(NBUF):
      wait_scat(bb)

    plsc.subcore_barrier()
    pltpu.sync_copy(acc_sh.at[pl.ds(s * ROWS_PT, ROWS_PT)],
                    out_hbm.at[pl.ds(c * N + s * ROWS_PT, ROWS_PT)])

    @pl.when(s == NTILES - 1)
    def _write_tail():
      pltpu.sync_copy(acc_sh.at[pl.ds(NTILES * ROWS_PT, ROWS_TAIL)],
                      out_hbm.at[pl.ds(c * N + NTILES * ROWS_PT, ROWS_TAIL)])

  return _sc_agg


def _aggregate(G, src, dst, zeros):
    return _sc_agg_kernel()(G, src, dst, zeros)


def _softmax_tables(u, t_i):
    p = u + EPS
    m = t_i * p
    cmax = jnp.max(m, axis=0, keepdims=True)
    g = jnp.exp(m - cmax)
    return g, g * p


def _mlp(z, W1, b1, bng, bnb, W2, b2):
    z = jnp.dot(z, W1, preferred_element_type=jnp.float32) + b1
    mu = jnp.mean(z, axis=0, keepdims=True)
    zc = z - mu
    var = jnp.mean(zc * zc, axis=0, keepdims=True)
    z = zc * lax.rsqrt(var + 1e-5) * bng + bnb
    z = jnp.maximum(z, 0.0)
    return jnp.dot(z, W2, preferred_element_type=jnp.float32) + b2


def _pre0_body(x_ref, t_ref, g2_ref):
    g, gp = _softmax_tables(jnp.maximum(x_ref[...], 0.0), t_ref[0])
    g2_ref[0] = g
    g2_ref[1] = gp


def _mid_body(u_ref, hb_ref, sesp_ref, W1_ref, b1_ref, bng_ref, bnb_ref,
              W2_ref, b2_ref, gng_ref, gnb_ref, gna_ref, t_ref,
              h_ref, u_out_ref, g2_ref, *, i):
    se = sesp_ref[0]
    sp = sesp_ref[1]
    z = u_ref[...] + sp / (se + 1e-16)
    cw = _mlp(z, W1_ref[i], b1_ref[i], bng_ref[i], bnb_ref[i],
              W2_ref[i], b2_ref[i])
    h = cw if i == 0 else hb_ref[...] + cw
    h_ref[...] = h
    mu = jnp.mean(h, axis=0, keepdims=True)
    hh = h - gna_ref[i] * mu
    var = jnp.mean(hh * hh, axis=0, keepdims=True)
    un = jnp.maximum(gng_ref[i] * hh * lax.rsqrt(var + 1e-5) + gnb_ref[i],
                     0.0)
    u_out_ref[...] = un
    g, gp = _softmax_tables(un, t_ref[i + 1])
    g2_ref[0] = g
    g2_ref[1] = gp


def _post_body(u_ref, hb_ref, sesp_ref, W1_ref, b1_ref, bng_ref, bnb_ref,
               W2_ref, b2_ref, LW_ref, Lb_ref, out_ref):
    se = sesp_ref[0]
    sp = sesp_ref[1]
    z = u_ref[...] + sp / (se + 1e-16)
    i = L - 1
    cw = _mlp(z, W1_ref[i], b1_ref[i], bng_ref[i], bnb_ref[i],
              W2_ref[i], b2_ref[i])
    h = hb_ref[...] + cw
    y = jnp.maximum(jnp.dot(h, LW_ref[0],
                            preferred_element_type=jnp.float32) + Lb_ref[0],
                    0.0)
    out_ref[...] = jnp.dot(y, LW_ref[1],
                           preferred_element_type=jnp.float32) + Lb_ref[1]


_f32 = lambda *s: jax.ShapeDtypeStruct(s, jnp.float32)


def _pre0(x, t):
    return pl.pallas_call(_pre0_body, out_shape=_f32(2, N, D))(x, t)


def _mid(i, u, hb, sesp, W1, b1, bn_g, bn_b, W2, b2, gn_g, gn_b, gn_a, t):
    body = functools.partial(_mid_body, i=i)
    return pl.pallas_call(
        body, out_shape=(_f32(N, D), _f32(N, D), _f32(2, N, D)))(
        u, hb, sesp, W1, b1, bn_g, bn_b, W2, b2, gn_g, gn_b, gn_a, t)


def _post(u, hb, sesp, W1, b1, bn_g, bn_b, W2, b2, LW, Lb):
    return pl.pallas_call(_post_body, out_shape=_f32(N, D))(
        u, hb, sesp, W1, b1, bn_g, bn_b, W2, b2, LW, Lb)


def kernel(x, edge_index, t, W1, b1, bn_g, bn_b, W2, b2,
           gn_g, gn_b, gn_a, LW, Lb):
    src = edge_index[0]
    dst = edge_index[1]
    zeros = jnp.zeros((ROWS_PT, D), jnp.float32)

    G = _pre0(x, t)
    u, h = x, x
    for i in range(L - 1):
        sesp = _aggregate(G.reshape(2 * N, D), src, dst, zeros).reshape(2, N, D)
        h, u, G = _mid(i, u, h, sesp, W1, b1, bn_g, bn_b, W2, b2,
                       gn_g, gn_b, gn_a, t)
    sesp = _aggregate(G.reshape(2 * N, D), src, dst, zeros).reshape(2, N, D)
    return _post(u, h, sesp, W1, b1, bn_g, bn_b, W2, b2, LW, Lb)


# speedup vs baseline: 1.0195x; 1.0195x over previous
import functools

import jax
import jax.numpy as jnp
from jax import lax
from jax.experimental import pallas as pl
from jax.experimental.pallas import tpu as pltpu
from jax.experimental.pallas import tpu_sc as plsc

N = 10000
E = 320000
D = 128
H = 2 * D
L = 4
NLIN = 2
EPS = 1e-7

NTILES = 16
EPT = E // NTILES
K = 128
NCHUNKS = E // K
NCH_BASE = NCHUNKS // NTILES
XTRA = NCHUNKS - NCH_BASE * NTILES
NBUF = 3
ROWS_PT = 624
ROWS_TAIL = N - NTILES * ROWS_PT

@functools.cache
def _sc_agg_kernel():
  mesh = plsc.VectorSubcoreMesh(core_axis_name="c", subcore_axis_name="s",
                                num_cores=2, num_subcores=NTILES)

  @functools.partial(
      pl.kernel,
      out_type=jax.ShapeDtypeStruct((2 * N, D), jnp.float32),
      mesh=mesh,
      scratch_types=[
        pltpu.VMEM((NBUF, K), jnp.int32),
        pltpu.VMEM((NBUF, K), jnp.int32),
        pltpu.VMEM((NBUF, K), jnp.int32),
        pltpu.VMEM((NBUF, K, D), jnp.float32),
        pltpu.VMEM_SHARED((N, D), jnp.float32),
        pltpu.SemaphoreType.DMA((NBUF,)),
        pltpu.SemaphoreType.DMA((NBUF,)),
        pltpu.SemaphoreType.DMA((NBUF,)),
        pltpu.SemaphoreType.DMA((NBUF,)),
      ],
  )
  def _sc_agg(tab_hbm, src_hbm, dst_hbm, zeros_hbm, out_hbm,
              idxd_v, gsrc_v, sdst_v, rows_v,
              acc_sh, sem_is, sem_id, sem_g, sem_w):
    c = lax.axis_index("c")
    s = lax.axis_index("s")
    nch = NCH_BASE + jnp.where(s < XTRA, 1, 0)
    base0 = s * (NCH_BASE * K) + jnp.minimum(s, XTRA) * K
    off = c * N

    def start_idx(j, b):
      o = base0 + j * K
      pltpu.async_copy(src_hbm.at[pl.ds(o, K)], gsrc_v.at[b], sem_is.at[b])
      pltpu.async_copy(dst_hbm.at[pl.ds(o, K)], idxd_v.at[b], sem_id.at[b])

    def wait_idx(j, b):
      o = base0 + j * K
      pltpu.make_async_copy(src_hbm.at[pl.ds(o, K)], gsrc_v.at[b],
                            sem_is.at[b]).wait()
      pltpu.make_async_copy(dst_hbm.at[pl.ds(o, K)], idxd_v.at[b],
                            sem_id.at[b]).wait()

    def copy_idx(b):
      gs, sd = gsrc_v.at[b], sdst_v.at[b]
      id_ = idxd_v.at[b]
      for k in range(K // 16):
        sl = pl.ds(k * 16, 16)
        gs[sl] = gs[sl] + off
        sd[sl] = id_[sl]

    def start_gather(b):
      pltpu.async_copy(tab_hbm.at[gsrc_v.at[b]], rows_v.at[b], sem_g.at[b])

    def wait_gather(b):
      pltpu.make_async_copy(tab_hbm.at[gsrc_v.at[b]], rows_v.at[b],
                            sem_g.at[b]).wait()

    def start_scat(b):
      pltpu.async_copy(rows_v.at[b], acc_sh.at[sdst_v.at[b]], sem_w.at[b],
                       add=True)

    def wait_scat(b):
      pltpu.make_async_copy(rows_v.at[b], acc_sh.at[sdst_v.at[b]],
                            sem_w.at[b]).wait()

    for jj in range(NBUF):
      start_idx(jj, jj)

    pltpu.sync_copy(zeros_hbm.at[pl.ds(0, ROWS_PT)],
                    acc_sh.at[pl.ds(s * ROWS_PT, ROWS_PT)])

    @pl.when(s == NTILES - 1)
    def _zero_tail():
      pltpu.sync_copy(zeros_hbm.at[pl.ds(0, ROWS_TAIL)],
                      acc_sh.at[pl.ds(NTILES * ROWS_PT, ROWS_TAIL)])

    plsc.subcore_barrier()

    @pl.loop(0, nch)
    def _chunk(j):
      b = lax.rem(j, NBUF)
      pb = lax.rem(j + NBUF - 1, NBUF)
      wait_idx(j, b)

      @pl.when(j >= NBUF)
      def _():
        wait_scat(b)

      copy_idx(b)
      start_gather(b)

      @pl.when(j >= 1)
      def _():
        wait_gather(pb)
        start_scat(pb)

        @pl.when(j - 1 + NBUF < nch)
        def _():
          start_idx(j - 1 + NBUF, pb)

    last = lax.rem(nch - 1, NBUF)
    wait_gather(last)
    start_scat(last)
    for bb in range(NBUF):
      wait_scat(bb)

    plsc.subcore_barrier()
    pltpu.sync_copy(acc_sh.at[pl.ds(s * ROWS_PT, ROWS_PT)],
                    out_hbm.at[pl.ds(c * N + s * ROWS_PT, ROWS_PT)])

    @pl.when(s == NTILES - 1)
    def _write_tail():
      pltpu.sync_copy(acc_sh.at[pl.ds(NTILES * ROWS_PT, ROWS_TAIL)],
                      out_hbm.at[pl.ds(c * N + NTILES * ROWS_PT, ROWS_TAIL)])

  return _sc_agg


def _aggregate(G, src, dst, zeros):
    return _sc_agg_kernel()(G, src, dst, zeros)


def _softmax_tables(u, t_i):
    p = u + EPS
    m = t_i * p
    cmax = jnp.max(m, axis=0, keepdims=True)
    g = jnp.exp(m - cmax)
    return g, g * p


def _mlp(z, W1, b1, bng, bnb, W2, b2):
    z = jnp.dot(z, W1, preferred_element_type=jnp.float32) + b1
    mu = jnp.mean(z, axis=0, keepdims=True)
    zc = z - mu
    var = jnp.mean(zc * zc, axis=0, keepdims=True)
    z = zc * lax.rsqrt(var + 1e-5) * bng + bnb
    z = jnp.maximum(z, 0.0)
    return jnp.dot(z, W2, preferred_element_type=jnp.float32) + b2


def _pre0_body(x_ref, t_ref, g2_ref):
    g, gp = _softmax_tables(jnp.maximum(x_ref[...], 0.0), t_ref[0])
    g2_ref[0] = g
    g2_ref[1] = gp


def _reconstruct_u(hb, st, gng, gnb):
    return jnp.maximum(gng * (hb - st[0]) * st[1] + gnb, 0.0)


def _mid_body(hb_ref, sesp_ref, st_ref, W1_ref, b1_ref, bng_ref, bnb_ref,
              W2_ref, b2_ref, gng_ref, gnb_ref, gna_ref, t_ref,
              h_ref, st_out_ref, g2_ref, *, i):
    hb = hb_ref[...]
    u = hb if i == 0 else _reconstruct_u(hb, st_ref, gng_ref[i - 1],
                                         gnb_ref[i - 1])
    se = sesp_ref[0]
    sp = sesp_ref[1]
    z = u + sp / (se + 1e-16)
    cw = _mlp(z, W1_ref[i], b1_ref[i], bng_ref[i], bnb_ref[i],
              W2_ref[i], b2_ref[i])
    h = cw if i == 0 else hb + cw
    h_ref[...] = h
    mu = jnp.mean(h, axis=0, keepdims=True)
    hh = h - gna_ref[i] * mu
    var = jnp.mean(hh * hh, axis=0, keepdims=True)
    rs = lax.rsqrt(var + 1e-5)
    st_out_ref[0] = (gna_ref[i] * mu)[0]
    st_out_ref[1] = rs[0]
    un = jnp.maximum(gng_ref[i] * hh * rs + gnb_ref[i], 0.0)
    g, gp = _softmax_tables(un, t_ref[i + 1])
    g2_ref[0] = g
    g2_ref[1] = gp


def _post_body(hb_ref, sesp_ref, st_ref, W1_ref, b1_ref, bng_ref, bnb_ref,
               W2_ref, b2_ref, gng_ref, gnb_ref, LW_ref, Lb_ref, out_ref):
    hb = hb_ref[...]
    u = _reconstruct_u(hb, st_ref, gng_ref[L - 2], gnb_ref[L - 2])
    se = sesp_ref[0]
    sp = sesp_ref[1]
    z = u + sp / (se + 1e-16)
    i = L - 1
    cw = _mlp(z, W1_ref[i], b1_ref[i], bng_ref[i], bnb_ref[i],
              W2_ref[i], b2_ref[i])
    h = hb + cw
    y = jnp.maximum(jnp.dot(h, LW_ref[0],
                            preferred_element_type=jnp.float32) + Lb_ref[0],
                    0.0)
    out_ref[...] = jnp.dot(y, LW_ref[1],
                           preferred_element_type=jnp.float32) + Lb_ref[1]


_f32 = lambda *s: jax.ShapeDtypeStruct(s, jnp.float32)


def _pre0(x, t):
    return pl.pallas_call(_pre0_body, out_shape=_f32(2, N, D))(x, t)


def _mid(i, hb, sesp, st, W1, b1, bn_g, bn_b, W2, b2, gn_g, gn_b, gn_a, t):
    body = functools.partial(_mid_body, i=i)
    return pl.pallas_call(
        body, out_shape=(_f32(N, D), _f32(2, D), _f32(2, N, D)))(
        hb, sesp, st, W1, b1, bn_g, bn_b, W2, b2, gn_g, gn_b, gn_a, t)


def _post(hb, sesp, st, W1, b1, bn_g, bn_b, W2, b2, gn_g, gn_b, LW, Lb):
    return pl.pallas_call(_post_body, out_shape=_f32(N, D))(
        hb, sesp, st, W1, b1, bn_g, bn_b, W2, b2, gn_g, gn_b, LW, Lb)


def kernel(x, edge_index, t, W1, b1, bn_g, bn_b, W2, b2,
           gn_g, gn_b, gn_a, LW, Lb):
    src = edge_index[0]
    dst = edge_index[1]
    zeros = jnp.zeros((ROWS_PT, D), jnp.float32)

    G = _pre0(x, t)
    h = x
    st = jnp.zeros((2, D), jnp.float32)
    for i in range(L - 1):
        sesp = _aggregate(G.reshape(2 * N, D), src, dst, zeros).reshape(2, N, D)
        h, st, G = _mid(i, h, sesp, st, W1, b1, bn_g, bn_b, W2, b2,
                        gn_g, gn_b, gn_a, t)
    sesp = _aggregate(G.reshape(2 * N, D), src, dst, zeros).reshape(2, N, D)
    return _post(h, sesp, st, W1, b1, bn_g, bn_b, W2, b2, gn_g, gn_b, LW, Lb)
